# Initial kernel scaffold; baseline (speedup 1.0000x reference)
import jax
import jax.numpy as jnp
from jax.experimental import pallas as pl


def kernel(phoneme_ids, table):
    raise NotImplementedError("write your pallas kernel here")


# speedup vs baseline: 4.1480x; 4.1480x over previous
import functools

import jax
import jax.numpy as jnp
from jax import lax
from jax.experimental import pallas as pl
from jax.experimental.pallas import tpu as pltpu
from jax.experimental.pallas import tpu_sc as plsc

EMBED_DIM = 64
NUM_CORES = 2
NUM_SUBCORES = 16
NUM_WORKERS = NUM_CORES * NUM_SUBCORES
CHUNK = 128
GATHERS_PER_STEP = 4
STEP = CHUNK * GATHERS_PER_STEP


def _emb_kernel(ids_hbm, table_hbm, out_hbm, idx_v, rows_v, sem):
    wid = lax.axis_index("s") * NUM_CORES + lax.axis_index("c")
    n_rows = ids_hbm.shape[0]
    rows_per_worker = n_rows // NUM_WORKERS
    steps = rows_per_worker // GATHERS_PER_STEP

    def body(g, carry):
        row0 = wid * rows_per_worker + g * GATHERS_PER_STEP
        tok0 = row0 * CHUNK
        pltpu.sync_copy(ids_hbm.at[pl.ds(row0, GATHERS_PER_STEP)], idx_v)
        copies = []
        for j in range(GATHERS_PER_STEP):
            copies.append(
                pltpu.async_copy(
                    table_hbm.at[idx_v.at[j]],
                    rows_v.at[pl.ds(j * CHUNK, CHUNK)],
                    sem,
                )
            )
        for c in copies:
            c.wait()
        pltpu.sync_copy(rows_v, out_hbm.at[pl.ds(tok0, STEP)])
        return carry

    lax.fori_loop(0, steps, body, 0)


def kernel(phoneme_ids, table):
    b, t = phoneme_ids.shape
    n = b * t
    ids2d = phoneme_ids.reshape(n // CHUNK, CHUNK).astype(jnp.int32)

    emb = functools.partial(
        pl.kernel,
        mesh=plsc.VectorSubcoreMesh(core_axis_name="c", subcore_axis_name="s"),
        out_type=jax.ShapeDtypeStruct((n, EMBED_DIM), jnp.float32),
        scratch_types=[
            pltpu.VMEM((GATHERS_PER_STEP, CHUNK), jnp.int32),
            pltpu.VMEM((STEP, EMBED_DIM), jnp.float32),
            pltpu.SemaphoreType.DMA,
        ],
        compiler_params=pltpu.CompilerParams(use_tc_tiling_on_sc=False),
    )(_emb_kernel)

    out = emb(ids2d, table)
    return out.reshape(b, t, EMBED_DIM)
